# baseline (speedup 1.0000x reference)
import functools

import jax
import jax.numpy as jnp
from jax import lax
from jax.experimental import pallas as pl
from jax.experimental.pallas import tpu as pltpu
from jax.experimental.pallas import tpu_sc as plsc

N = 4096
D = 1024
H = 2048
E = 8
S = 2
T = 256
NT = 40
NPAD = NT * T

TSH = 512
TCB = 512

SC_CORES = 2
SC_SUBCORES = 16
NW = SC_CORES * SC_SUBCORES
TPW = N // NW
CH = 16
NCH = TPW // CH


def _route_body(e1_ref, e2_ref, v1_ref, v2_ref, p1_ref, p2_ref, g1_ref,
                g2_ref, meta_ref):
    iota_e = lax.broadcasted_iota(jnp.int32, (E, N), 0)
    e1 = e1_ref[...]
    e2 = e2_ref[...]
    v1 = v1_ref[...]
    v2 = v2_ref[...]

    ex = jnp.exp(v2 - v1)
    den = 1.0 + ex
    g1_ref[...] = 1.0 / den
    g2_ref[...] = ex / den

    sel = (jnp.logical_or(iota_e == e1, iota_e == e2)).astype(jnp.int32)
    cum = sel
    k = 1
    while k < N:
        cum = cum + jnp.concatenate(
            [jnp.zeros((E, k), jnp.int32), cum[:, :N - k]], axis=1)
        k *= 2
    counts = cum[:, N - 1:N]
    pc = (counts + (T - 1)) // T
    incl = pc
    k = 1
    while k < E:
        incl = incl + jnp.concatenate(
            [jnp.zeros((k, 1), jnp.int32), incl[:E - k, :]], axis=0)
        k *= 2
    texcl = incl - pc
    pos = texcl * T + cum - 1
    p1_ref[...] = jnp.sum(jnp.where(iota_e == e1, pos, 0), axis=0,
                          keepdims=True)
    p2_ref[...] = jnp.sum(jnp.where(iota_e == e2, pos, 0), axis=0,
                          keepdims=True)

    total = incl[E - 1:E, :]
    iota_t = lax.broadcasted_iota(jnp.int32, (1, NT), 1)
    te = jnp.sum((iota_t >= texcl).astype(jnp.int32), axis=0,
                 keepdims=True) - 1
    tvalid = (iota_t < total).astype(jnp.int32)
    meta_ref[...] = jnp.concatenate([te, tvalid], axis=0)


def _route(e1, e2, v1, v2):
    return pl.pallas_call(
        _route_body,
        out_shape=[
            jax.ShapeDtypeStruct((1, N), jnp.int32),
            jax.ShapeDtypeStruct((1, N), jnp.int32),
            jax.ShapeDtypeStruct((1, N), jnp.float32),
            jax.ShapeDtypeStruct((1, N), jnp.float32),
            jax.ShapeDtypeStruct((2, NT), jnp.int32),
        ],
    )(e1, e2, v1, v2)


def _dispatch_body(x_hbm, p1_hbm, p2_hbm, xs_hbm, idx1_v, idx2_v, rows_a,
                   rows_b, semr, sems):
    wid = lax.axis_index("s") * SC_CORES + lax.axis_index("c")
    pltpu.sync_copy(p1_hbm.at[pl.ds(wid * NCH, NCH)], idx1_v)
    pltpu.sync_copy(p2_hbm.at[pl.ds(wid * NCH, NCH)], idx2_v)
    rows = (rows_a, rows_b)
    c_load = pltpu.async_copy(x_hbm.at[pl.ds(wid * TPW, CH)], rows_a, semr)
    prev = None
    for j in range(NCH):
        cur = rows[j % 2]
        c_load.wait()
        if prev is not None:
            prev[0].wait()
            prev[1].wait()
        if j + 1 < NCH:
            c_load = pltpu.async_copy(
                x_hbm.at[pl.ds(wid * TPW + (j + 1) * CH, CH)],
                rows[(j + 1) % 2], semr)
        s1 = pltpu.async_copy(cur, xs_hbm.at[idx1_v.at[j]], sems)
        s2 = pltpu.async_copy(cur, xs_hbm.at[idx2_v.at[j]], sems)
        prev = (s1, s2)
    prev[0].wait()
    prev[1].wait()


def _dispatch(xf, pos1, pos2):
    mesh = plsc.VectorSubcoreMesh(core_axis_name="c", subcore_axis_name="s")
    return pl.kernel(
        _dispatch_body,
        out_type=jax.ShapeDtypeStruct((NPAD, D), jnp.float32),
        mesh=mesh,
        scratch_types=[
            pltpu.VMEM((NCH, CH), jnp.int32),
            pltpu.VMEM((NCH, CH), jnp.int32),
            pltpu.VMEM((CH, D), jnp.float32),
            pltpu.VMEM((CH, D), jnp.float32),
            pltpu.SemaphoreType.DMA,
            pltpu.SemaphoreType.DMA,
        ],
    )(xf, pos1, pos2)


def _shared_body(x_ref, w1_ref, w3_ref, w2_ref, o_ref):
    j = pl.program_id(1)
    xb = x_ref[...]
    h1 = jnp.dot(xb, w1_ref[0], preferred_element_type=jnp.float32)
    h3 = jnp.dot(xb, w3_ref[0], preferred_element_type=jnp.float32)
    h = h1 * jax.nn.sigmoid(h1) * h3
    yt = jnp.dot(h, w2_ref[0], preferred_element_type=jnp.float32)

    @pl.when(j == 0)
    def _():
        o_ref[...] = yt

    @pl.when(j != 0)
    def _():
        o_ref[...] = o_ref[...] + yt


def _shared(xf, sw1b, sw3b, sw2b):
    return pl.pallas_call(
        _shared_body,
        grid=(N // TSH, S * 2),
        in_specs=[
            pl.BlockSpec((TSH, D), lambda i, j: (i, 0)),
            pl.BlockSpec((1, D, H // 2), lambda i, j: (j // 2, 0, j % 2)),
            pl.BlockSpec((1, D, H // 2), lambda i, j: (j // 2, 0, j % 2)),
            pl.BlockSpec((1, H // 2, D), lambda i, j: (j // 2, j % 2, 0)),
        ],
        out_specs=pl.BlockSpec((TSH, D), lambda i, j: (i, 0)),
        out_shape=jax.ShapeDtypeStruct((N, D), jnp.float32),
        compiler_params=pltpu.CompilerParams(
            dimension_semantics=("parallel", "arbitrary")),
    )(xf, sw1b, sw3b, sw2b)


def _ffn_body(meta_ref, xs_ref, w1_ref, w3_ref, w2_ref, y_ref):
    i = pl.program_id(0)
    j = pl.program_id(1)

    @pl.when(meta_ref[1, i] == 1)
    def _():
        xb = xs_ref[...]
        h1 = jnp.dot(xb, w1_ref[0], preferred_element_type=jnp.float32)
        h3 = jnp.dot(xb, w3_ref[0], preferred_element_type=jnp.float32)
        h = h1 * jax.nn.sigmoid(h1) * h3
        yt = jnp.dot(h, w2_ref[0], preferred_element_type=jnp.float32)

        @pl.when(j == 0)
        def _():
            y_ref[...] = yt

        @pl.when(j != 0)
        def _():
            y_ref[...] = y_ref[...] + yt


def _ffn(meta, xs, rw1b, rw3b, rw2b):
    grid_spec = pltpu.PrefetchScalarGridSpec(
        num_scalar_prefetch=1,
        grid=(NT, 2),
        in_specs=[
            pl.BlockSpec((T, D), lambda i, j, m: (i, 0)),
            pl.BlockSpec((1, D, H // 2), lambda i, j, m: (m[0, i], 0, j)),
            pl.BlockSpec((1, D, H // 2), lambda i, j, m: (m[0, i], 0, j)),
            pl.BlockSpec((1, H // 2, D), lambda i, j, m: (m[0, i], j, 0)),
        ],
        out_specs=pl.BlockSpec((T, D), lambda i, j, m: (i, 0)),
    )
    return pl.pallas_call(
        _ffn_body,
        grid_spec=grid_spec,
        out_shape=jax.ShapeDtypeStruct((NPAD, D), jnp.float32),
        compiler_params=pltpu.CompilerParams(
            dimension_semantics=("parallel", "arbitrary")),
    )(meta, xs, rw1b, rw3b, rw2b)


def _gather_body(y_hbm, p1_hbm, p2_hbm, o1_hbm, o2_hbm, idx1_v, idx2_v,
                 r1_a, r1_b, r2_a, r2_b, semg, semw):
    wid = lax.axis_index("s") * SC_CORES + lax.axis_index("c")
    pltpu.sync_copy(p1_hbm.at[pl.ds(wid * NCH, NCH)], idx1_v)
    pltpu.sync_copy(p2_hbm.at[pl.ds(wid * NCH, NCH)], idx2_v)
    r1 = (r1_a, r1_b)
    r2 = (r2_a, r2_b)
    g1 = pltpu.async_copy(y_hbm.at[idx1_v.at[0]], r1_a, semg)
    g2 = pltpu.async_copy(y_hbm.at[idx2_v.at[0]], r2_a, semg)
    prev = None
    for j in range(NCH):
        base = wid * TPW + j * CH
        g1.wait()
        g2.wait()
        if prev is not None:
            prev[0].wait()
            prev[1].wait()
        if j + 1 < NCH:
            g1 = pltpu.async_copy(y_hbm.at[idx1_v.at[j + 1]],
                                  r1[(j + 1) % 2], semg)
            g2 = pltpu.async_copy(y_hbm.at[idx2_v.at[j + 1]],
                                  r2[(j + 1) % 2], semg)
        w1 = pltpu.async_copy(r1[j % 2], o1_hbm.at[pl.ds(base, CH)], semw)
        w2 = pltpu.async_copy(r2[j % 2], o2_hbm.at[pl.ds(base, CH)], semw)
        prev = (w1, w2)
    prev[0].wait()
    prev[1].wait()


def _gather(y, pos1, pos2):
    mesh = plsc.VectorSubcoreMesh(core_axis_name="c", subcore_axis_name="s")
    return pl.kernel(
        _gather_body,
        out_type=[
            jax.ShapeDtypeStruct((N, D), jnp.float32),
            jax.ShapeDtypeStruct((N, D), jnp.float32),
        ],
        mesh=mesh,
        scratch_types=[
            pltpu.VMEM((NCH, CH), jnp.int32),
            pltpu.VMEM((NCH, CH), jnp.int32),
            pltpu.VMEM((CH, D), jnp.float32),
            pltpu.VMEM((CH, D), jnp.float32),
            pltpu.VMEM((CH, D), jnp.float32),
            pltpu.VMEM((CH, D), jnp.float32),
            pltpu.SemaphoreType.DMA,
            pltpu.SemaphoreType.DMA,
        ],
    )(y, pos1, pos2)


def _combine_body(sh_ref, y1_ref, y2_ref, g1_ref, g2_ref, o_ref):
    o_ref[...] = (sh_ref[...] + y1_ref[...] * g1_ref[...]
                  + y2_ref[...] * g2_ref[...])


def _combine(shared, y1, y2, g1c, g2c):
    return pl.pallas_call(
        _combine_body,
        grid=(N // TCB,),
        in_specs=[
            pl.BlockSpec((TCB, D), lambda i: (i, 0)),
            pl.BlockSpec((TCB, D), lambda i: (i, 0)),
            pl.BlockSpec((TCB, D), lambda i: (i, 0)),
            pl.BlockSpec((TCB, 1), lambda i: (i, 0)),
            pl.BlockSpec((TCB, 1), lambda i: (i, 0)),
        ],
        out_specs=pl.BlockSpec((TCB, D), lambda i: (i, 0)),
        out_shape=jax.ShapeDtypeStruct((N, D), jnp.float32),
        compiler_params=pltpu.CompilerParams(
            dimension_semantics=("parallel",)),
    )(shared, y1, y2, g1c, g2c)


@jax.jit
def kernel(x, Wg, bg, Wn, bn, sw1, sw2, sw3, rw1, rw2, rw3):
    b, s, d = x.shape
    xf = x.reshape(N, D)
    logits = xf @ Wg.T + bg
    topv, idxs = jax.lax.top_k(logits, 2)
    pos1, pos2, g1, g2, meta = _route(
        idxs[:, 0].reshape(1, N), idxs[:, 1].reshape(1, N),
        topv[:, 0].reshape(1, N), topv[:, 1].reshape(1, N))
    p1c = pos1.reshape(NW * NCH, CH)
    p2c = pos2.reshape(NW * NCH, CH)
    xs = _dispatch(xf, p1c, p2c)
    shared = _shared(xf, sw1, sw3, sw2)
    y = _ffn(meta, xs, rw1, rw3, rw2)
    y1, y2 = _gather(y, p1c, p2c)
    out = _combine(shared, y1, y2, g1.reshape(N, 1), g2.reshape(N, 1))
    return out.reshape(b, s, d)


# speedup vs baseline: 1.3415x; 1.3415x over previous
import functools

import jax
import jax.numpy as jnp
from jax import lax
from jax.experimental import pallas as pl
from jax.experimental.pallas import tpu as pltpu
from jax.experimental.pallas import tpu_sc as plsc

N = 4096
D = 1024
H = 2048
E = 8
S = 2
T = 256
NT = 40
NPAD = NT * T

TSH = 512
TCB = 512

SC_CORES = 2
SC_SUBCORES = 16
NW = SC_CORES * SC_SUBCORES
TPW = N // NW
CH = 16
NCH = TPW // CH


def _route_body(e1_ref, e2_ref, v1_ref, v2_ref, p1_ref, p2_ref, g1_ref,
                g2_ref, meta_ref):
    iota_e = lax.broadcasted_iota(jnp.int32, (E, N), 0)
    e1 = e1_ref[...]
    e2 = e2_ref[...]
    v1 = v1_ref[...]
    v2 = v2_ref[...]

    ex = jnp.exp(v2 - v1)
    den = 1.0 + ex
    g1_ref[...] = 1.0 / den
    g2_ref[...] = ex / den

    sel = (jnp.logical_or(iota_e == e1, iota_e == e2)).astype(jnp.int32)
    cum = sel
    k = 1
    while k < N:
        cum = cum + jnp.concatenate(
            [jnp.zeros((E, k), jnp.int32), cum[:, :N - k]], axis=1)
        k *= 2
    counts = cum[:, N - 1:N]
    pc = (counts + (T - 1)) // T
    incl = pc
    k = 1
    while k < E:
        incl = incl + jnp.concatenate(
            [jnp.zeros((k, 1), jnp.int32), incl[:E - k, :]], axis=0)
        k *= 2
    texcl = incl - pc
    pos = texcl * T + cum - 1
    p1_ref[...] = jnp.sum(jnp.where(iota_e == e1, pos, 0), axis=0,
                          keepdims=True)
    p2_ref[...] = jnp.sum(jnp.where(iota_e == e2, pos, 0), axis=0,
                          keepdims=True)

    total = incl[E - 1:E, :]
    iota_t = lax.broadcasted_iota(jnp.int32, (1, NT), 1)
    te = jnp.sum((iota_t >= texcl).astype(jnp.int32), axis=0,
                 keepdims=True) - 1
    tvalid = (iota_t < total).astype(jnp.int32)
    meta_ref[...] = jnp.concatenate([te, tvalid], axis=0)


def _route(e1, e2, v1, v2):
    return pl.pallas_call(
        _route_body,
        out_shape=[
            jax.ShapeDtypeStruct((1, N), jnp.int32),
            jax.ShapeDtypeStruct((1, N), jnp.int32),
            jax.ShapeDtypeStruct((1, N), jnp.float32),
            jax.ShapeDtypeStruct((1, N), jnp.float32),
            jax.ShapeDtypeStruct((2, NT), jnp.int32),
        ],
    )(e1, e2, v1, v2)


def _dispatch_body(x_hbm, p1_hbm, p2_hbm, xs_hbm, idx1_v, idx2_v, rows_a,
                   rows_b, semr, sems):
    wid = lax.axis_index("s") * SC_CORES + lax.axis_index("c")
    pltpu.sync_copy(p1_hbm.at[pl.ds(wid * NCH, NCH)], idx1_v)
    pltpu.sync_copy(p2_hbm.at[pl.ds(wid * NCH, NCH)], idx2_v)
    rows = (rows_a, rows_b)
    c_load = pltpu.async_copy(x_hbm.at[pl.ds(wid * TPW, CH)], rows_a, semr)
    prev = None
    for j in range(NCH):
        cur = rows[j % 2]
        c_load.wait()
        if prev is not None:
            prev[0].wait()
            prev[1].wait()
        if j + 1 < NCH:
            c_load = pltpu.async_copy(
                x_hbm.at[pl.ds(wid * TPW + (j + 1) * CH, CH)],
                rows[(j + 1) % 2], semr)
        s1 = pltpu.async_copy(cur, xs_hbm.at[idx1_v.at[j]], sems)
        s2 = pltpu.async_copy(cur, xs_hbm.at[idx2_v.at[j]], sems)
        prev = (s1, s2)
    prev[0].wait()
    prev[1].wait()


def _dispatch(xf, pos1, pos2):
    mesh = plsc.VectorSubcoreMesh(core_axis_name="c", subcore_axis_name="s")
    return pl.kernel(
        _dispatch_body,
        out_type=jax.ShapeDtypeStruct((NPAD, D), jnp.float32),
        mesh=mesh,
        scratch_types=[
            pltpu.VMEM((NCH, CH), jnp.int32),
            pltpu.VMEM((NCH, CH), jnp.int32),
            pltpu.VMEM((CH, D), jnp.float32),
            pltpu.VMEM((CH, D), jnp.float32),
            pltpu.SemaphoreType.DMA,
            pltpu.SemaphoreType.DMA,
        ],
    )(xf, pos1, pos2)


def _shared_body(x_ref, w1_ref, w3_ref, w2_ref, o_ref):
    j = pl.program_id(1)
    xb = x_ref[...]
    h1 = jnp.dot(xb, w1_ref[0], preferred_element_type=jnp.float32)
    h3 = jnp.dot(xb, w3_ref[0], preferred_element_type=jnp.float32)
    h = h1 * jax.nn.sigmoid(h1) * h3
    yt = jnp.dot(h, w2_ref[0], preferred_element_type=jnp.float32)

    @pl.when(j == 0)
    def _():
        o_ref[...] = yt

    @pl.when(j != 0)
    def _():
        o_ref[...] = o_ref[...] + yt


def _shared(xf, sw1b, sw3b, sw2b):
    return pl.pallas_call(
        _shared_body,
        grid=(N // TSH, S),
        in_specs=[
            pl.BlockSpec((TSH, D), lambda i, j: (i, 0)),
            pl.BlockSpec((1, D, H), lambda i, j: (j, 0, 0)),
            pl.BlockSpec((1, D, H), lambda i, j: (j, 0, 0)),
            pl.BlockSpec((1, H, D), lambda i, j: (j, 0, 0)),
        ],
        out_specs=pl.BlockSpec((TSH, D), lambda i, j: (i, 0)),
        out_shape=jax.ShapeDtypeStruct((N, D), jnp.float32),
        compiler_params=pltpu.CompilerParams(
            dimension_semantics=("parallel", "arbitrary"),
            vmem_limit_bytes=67108864),
    )(xf, sw1b, sw3b, sw2b)


def _ffn_body(meta_ref, xs_ref, w1_ref, w3_ref, w2_ref, y_ref):
    i = pl.program_id(0)

    @pl.when(meta_ref[1, i] == 1)
    def _():
        xb = xs_ref[...]
        h1 = jnp.dot(xb, w1_ref[0], preferred_element_type=jnp.float32)
        h3 = jnp.dot(xb, w3_ref[0], preferred_element_type=jnp.float32)
        h = h1 * jax.nn.sigmoid(h1) * h3
        y_ref[...] = jnp.dot(h, w2_ref[0], preferred_element_type=jnp.float32)


def _ffn(meta, xs, rw1b, rw3b, rw2b):
    grid_spec = pltpu.PrefetchScalarGridSpec(
        num_scalar_prefetch=1,
        grid=(NT,),
        in_specs=[
            pl.BlockSpec((T, D), lambda i, m: (i, 0)),
            pl.BlockSpec((1, D, H), lambda i, m: (m[0, i], 0, 0)),
            pl.BlockSpec((1, D, H), lambda i, m: (m[0, i], 0, 0)),
            pl.BlockSpec((1, H, D), lambda i, m: (m[0, i], 0, 0)),
        ],
        out_specs=pl.BlockSpec((T, D), lambda i, m: (i, 0)),
    )
    return pl.pallas_call(
        _ffn_body,
        grid_spec=grid_spec,
        out_shape=jax.ShapeDtypeStruct((NPAD, D), jnp.float32),
        compiler_params=pltpu.CompilerParams(
            dimension_semantics=("parallel",),
            vmem_limit_bytes=67108864),
    )(meta, xs, rw1b, rw3b, rw2b)


def _gather_body(y_hbm, p1_hbm, p2_hbm, o1_hbm, o2_hbm, idx1_v, idx2_v,
                 r1_a, r1_b, r2_a, r2_b, semg, semw):
    wid = lax.axis_index("s") * SC_CORES + lax.axis_index("c")
    pltpu.sync_copy(p1_hbm.at[pl.ds(wid * NCH, NCH)], idx1_v)
    pltpu.sync_copy(p2_hbm.at[pl.ds(wid * NCH, NCH)], idx2_v)
    r1 = (r1_a, r1_b)
    r2 = (r2_a, r2_b)
    g1 = pltpu.async_copy(y_hbm.at[idx1_v.at[0]], r1_a, semg)
    g2 = pltpu.async_copy(y_hbm.at[idx2_v.at[0]], r2_a, semg)
    prev = None
    for j in range(NCH):
        base = wid * TPW + j * CH
        g1.wait()
        g2.wait()
        if prev is not None:
            prev[0].wait()
            prev[1].wait()
        if j + 1 < NCH:
            g1 = pltpu.async_copy(y_hbm.at[idx1_v.at[j + 1]],
                                  r1[(j + 1) % 2], semg)
            g2 = pltpu.async_copy(y_hbm.at[idx2_v.at[j + 1]],
                                  r2[(j + 1) % 2], semg)
        w1 = pltpu.async_copy(r1[j % 2], o1_hbm.at[pl.ds(base, CH)], semw)
        w2 = pltpu.async_copy(r2[j % 2], o2_hbm.at[pl.ds(base, CH)], semw)
        prev = (w1, w2)
    prev[0].wait()
    prev[1].wait()


def _gather(y, pos1, pos2):
    mesh = plsc.VectorSubcoreMesh(core_axis_name="c", subcore_axis_name="s")
    return pl.kernel(
        _gather_body,
        out_type=[
            jax.ShapeDtypeStruct((N, D), jnp.float32),
            jax.ShapeDtypeStruct((N, D), jnp.float32),
        ],
        mesh=mesh,
        scratch_types=[
            pltpu.VMEM((NCH, CH), jnp.int32),
            pltpu.VMEM((NCH, CH), jnp.int32),
            pltpu.VMEM((CH, D), jnp.float32),
            pltpu.VMEM((CH, D), jnp.float32),
            pltpu.VMEM((CH, D), jnp.float32),
            pltpu.VMEM((CH, D), jnp.float32),
            pltpu.SemaphoreType.DMA,
            pltpu.SemaphoreType.DMA,
        ],
    )(y, pos1, pos2)


def _combine_body(sh_ref, y1_ref, y2_ref, g1_ref, g2_ref, o_ref):
    o_ref[...] = (sh_ref[...] + y1_ref[...] * g1_ref[...]
                  + y2_ref[...] * g2_ref[...])


def _combine(shared, y1, y2, g1c, g2c):
    return pl.pallas_call(
        _combine_body,
        grid=(N // TCB,),
        in_specs=[
            pl.BlockSpec((TCB, D), lambda i: (i, 0)),
            pl.BlockSpec((TCB, D), lambda i: (i, 0)),
            pl.BlockSpec((TCB, D), lambda i: (i, 0)),
            pl.BlockSpec((TCB, 1), lambda i: (i, 0)),
            pl.BlockSpec((TCB, 1), lambda i: (i, 0)),
        ],
        out_specs=pl.BlockSpec((TCB, D), lambda i: (i, 0)),
        out_shape=jax.ShapeDtypeStruct((N, D), jnp.float32),
        compiler_params=pltpu.CompilerParams(
            dimension_semantics=("parallel",)),
    )(shared, y1, y2, g1c, g2c)


@jax.jit
def kernel(x, Wg, bg, Wn, bn, sw1, sw2, sw3, rw1, rw2, rw3):
    b, s, d = x.shape
    xf = x.reshape(N, D)
    logits = xf @ Wg.T + bg
    topv, idxs = jax.lax.top_k(logits, 2)
    pos1, pos2, g1, g2, meta = _route(
        idxs[:, 0].reshape(1, N), idxs[:, 1].reshape(1, N),
        topv[:, 0].reshape(1, N), topv[:, 1].reshape(1, N))
    p1c = pos1.reshape(NW * NCH, CH)
    p2c = pos2.reshape(NW * NCH, CH)
    xs = _dispatch(xf, p1c, p2c)
    shared = _shared(xf, sw1, sw3, sw2)
    y = _ffn(meta, xs, rw1, rw3, rw2)
    y1, y2 = _gather(y, p1c, p2c)
    out = _combine(shared, y1, y2, g1.reshape(N, 1), g2.reshape(N, 1))
    return out.reshape(b, s, d)
